# baseline (speedup 1.0000x reference)
import dataclasses
import functools

import jax
import jax.numpy as jnp
from jax import lax
from jax.experimental import pallas as pl
from jax.experimental.pallas import tpu as pltpu
from jax.experimental.pallas import tpu_sc as plsc

N = 100000
E = 6400000
NUM_ITER = 10
D = 0.85

NPAD = 100096
ACC = 100864
TRASH = 100800
NW = 32
WPT = 204800
EPAD = NW * WPT
CHUNK = 2560
R = CHUNK // 128
NCHUNK = WPT // CHUNK
ZB = 1600

_mesh = plsc.VectorSubcoreMesh(core_axis_name="c", subcore_axis_name="s")

_cp = pltpu.CompilerParams()
if "needs_layout_passes" in pltpu.CompilerParams.__dataclass_fields__:
    _cp = dataclasses.replace(_cp, needs_layout_passes=False)


@functools.partial(
    pl.kernel,
    out_type=jax.ShapeDtypeStruct((2 * NPAD,), jnp.float32),
    mesh=_mesh,
    compiler_params=_cp,
    scratch_types=[
        pltpu.VMEM((N,), jnp.float32),
        pltpu.VMEM((2, CHUNK), jnp.int32),
        pltpu.VMEM((2, R, 128), jnp.int32),
        pltpu.VMEM((2, CHUNK), jnp.float32),
        pltpu.VMEM((ZB,), jnp.float32),
        pltpu.VMEM_SHARED((ACC,), jnp.float32),
        pltpu.SemaphoreType.DMA,
        pltpu.SemaphoreType.DMA,
        pltpu.SemaphoreType.DMA,
        pltpu.SemaphoreType.DMA,
    ],
)
def _sc_iter(ni_hbm, src_hbm, tgt_hbm, out_hbm,
             ni_ts, tgtbuf, srcbuf, vals, zbuf, acc_sh,
             sem_in0, sem_in1, sem_sc0, sem_sc1):
    cid = lax.axis_index("c")
    tid = lax.axis_index("s")
    wid = cid * 16 + tid
    sem_in = (sem_in0, sem_in1)
    sem_sc = (sem_sc0, sem_sc1)

    for j in range(ZB // 16):
        zbuf[pl.ds(j * 16, 16)] = jnp.zeros((16,), jnp.float32)
    zbase = tid * (ACC // 16)
    for q in range(3):
        pltpu.sync_copy(zbuf, acc_sh.at[pl.ds(zbase + q * ZB, ZB)])
    pltpu.sync_copy(zbuf.at[pl.ds(0, 1504)],
                    acc_sh.at[pl.ds(zbase + 3 * ZB, 1504)])

    pltpu.sync_copy(ni_hbm.at[pl.ds(0, N)], ni_ts)
    plsc.subcore_barrier()

    def in_copies(n, b):
        off = wid * WPT + n * CHUNK
        return (
            pltpu.make_async_copy(tgt_hbm.at[pl.ds(off, CHUNK)],
                                  tgtbuf.at[b], sem_in[b]),
            pltpu.make_async_copy(src_hbm.at[wid * NCHUNK + n],
                                  srcbuf.at[b], sem_in[b]),
        )

    def gather(b):
        @plsc.parallel_loop(0, R, unroll=4)
        def _row(j):
            for k in range(8):
                idx = tgtbuf[b, pl.ds(j * 128 + k * 16, 16)]
                vals[b, pl.ds(j * 128 + k * 16, 16)] = plsc.load_gather(ni_ts, [idx])

    def fire_scatters(b):
        for j in range(R):
            pltpu.async_copy(vals.at[b, pl.ds(j * 128, 128)], acc_sh.at[srcbuf.at[b, j]],
                             sem_sc[b], add=True)

    def drain_scatters(b):
        for j in range(R):
            pltpu.make_async_copy(vals.at[b, pl.ds(j * 128, 128)], acc_sh.at[srcbuf.at[b, j]],
                                  sem_sc[b]).wait()

    def start_in(n, b):
        for cp in in_copies(n, b):
            cp.start()

    def wait_in(n, b):
        for cp in in_copies(n, b):
            cp.wait()

    start_in(0, 0)

    @pl.loop(0, NCHUNK // 2)
    def _pair(p):
        n0 = p * 2
        wait_in(n0, 0)
        gather(0)

        @pl.when(p > 0)
        def _():
            drain_scatters(1)

        start_in(n0 + 1, 1)
        fire_scatters(0)

        wait_in(n0 + 1, 1)
        gather(1)
        drain_scatters(0)

        @pl.when(p < NCHUNK // 2 - 1)
        def _():
            start_in(n0 + 2, 0)

        fire_scatters(1)

    drain_scatters(1)

    plsc.subcore_barrier()
    opt = NPAD // 16
    base = tid * opt
    for q in range(3):
        pltpu.sync_copy(acc_sh.at[pl.ds(base + q * ZB, ZB)], zbuf)
        pltpu.sync_copy(zbuf,
                        out_hbm.at[pl.ds(cid * NPAD + base + q * ZB, ZB)])
    pltpu.sync_copy(acc_sh.at[pl.ds(base + 3 * ZB, 1456)],
                    zbuf.at[pl.ds(0, 1456)])
    pltpu.sync_copy(zbuf.at[pl.ds(0, 1456)],
                    out_hbm.at[pl.ds(cid * NPAD + base + 3 * ZB, 1456)])


def _tc_norm(ni_pad, acc):
    rows = NPAD // 128

    def body(ni_ref, acc_ref, out_ref):
        norm = jnp.sum(ni_ref[...])
        s = D / norm
        v = (acc_ref[0] + acc_ref[1]) * s + (1.0 - D) / N
        r = lax.broadcasted_iota(jnp.int32, (rows, 128), 0)
        c = lax.broadcasted_iota(jnp.int32, (rows, 128), 1)
        out_ref[...] = jnp.where(r * 128 + c < N, v, 0.0)

    out = pl.pallas_call(
        body,
        out_shape=jax.ShapeDtypeStruct((rows, 128), jnp.float32),
    )(ni_pad.reshape(rows, 128), acc.reshape(2, rows, 128))
    return out.reshape(NPAD)


def kernel(node_influence, source_indices, target_indices):
    ni = jnp.zeros((NPAD,), jnp.float32).at[:N].set(node_influence)
    pad = EPAD - E
    src_p = jnp.concatenate(
        [source_indices, jnp.full((pad,), TRASH, jnp.int32)]
    ).reshape(EPAD // CHUNK, R, 128)
    tgt_p = jnp.concatenate([target_indices, jnp.zeros((pad,), jnp.int32)])
    for _ in range(NUM_ITER):
        acc = _sc_iter(ni, src_p, tgt_p)
        ni = _tc_norm(ni, acc)
    return ni[:N]


# speedup vs baseline: 1.0184x; 1.0184x over previous
import dataclasses
import functools

import jax
import jax.numpy as jnp
from jax import lax
from jax.experimental import pallas as pl
from jax.experimental.pallas import tpu as pltpu
from jax.experimental.pallas import tpu_sc as plsc

N = 100000
E = 6400000
NUM_ITER = 10
D = 0.85

NPAD = 100096
ACC = 100864
TRASH = 100800
NW = 32
WPT = 204800
EPAD = NW * WPT
CHUNK = 2048
R = CHUNK // 128
NCHUNK = WPT // CHUNK
ZB = 1600

_mesh = plsc.VectorSubcoreMesh(core_axis_name="c", subcore_axis_name="s")

_cp = pltpu.CompilerParams()
if "needs_layout_passes" in pltpu.CompilerParams.__dataclass_fields__:
    _cp = dataclasses.replace(_cp, needs_layout_passes=False)


@functools.partial(
    pl.kernel,
    out_type=jax.ShapeDtypeStruct((2 * NPAD,), jnp.float32),
    mesh=_mesh,
    compiler_params=_cp,
    scratch_types=[
        pltpu.VMEM((N,), jnp.float32),
        pltpu.VMEM((CHUNK,), jnp.int32),
        pltpu.VMEM((CHUNK,), jnp.int32),
        pltpu.VMEM((CHUNK,), jnp.int32),
        pltpu.VMEM((R, 128), jnp.int32),
        pltpu.VMEM((R, 128), jnp.int32),
        pltpu.VMEM((R, 128), jnp.int32),
        pltpu.VMEM((CHUNK,), jnp.float32),
        pltpu.VMEM((CHUNK,), jnp.float32),
        pltpu.VMEM((CHUNK,), jnp.float32),
        pltpu.VMEM((ZB,), jnp.float32),
        pltpu.VMEM_SHARED((ACC,), jnp.float32),
        pltpu.SemaphoreType.DMA,
        pltpu.SemaphoreType.DMA,
        pltpu.SemaphoreType.DMA,
        pltpu.SemaphoreType.DMA,
        pltpu.SemaphoreType.DMA,
        pltpu.SemaphoreType.DMA,
    ],
)
def _sc_iter(ni_hbm, src_hbm, tgt_hbm, out_hbm,
             ni_ts, tgt0, tgt1, tgt2, src0, src1, src2, val0, val1, val2,
             zbuf, acc_sh,
             sem_in0, sem_in1, sem_in2, sem_sc0, sem_sc1, sem_sc2):
    cid = lax.axis_index("c")
    tid = lax.axis_index("s")
    wid = cid * 16 + tid
    sem_in = (sem_in0, sem_in1, sem_in2)
    sem_sc = (sem_sc0, sem_sc1, sem_sc2)
    tgtbuf = (tgt0, tgt1, tgt2)
    srcbuf = (src0, src1, src2)
    vals = (val0, val1, val2)

    for j in range(ZB // 16):
        zbuf[pl.ds(j * 16, 16)] = jnp.zeros((16,), jnp.float32)
    zbase = tid * (ACC // 16)
    for q in range(3):
        pltpu.sync_copy(zbuf, acc_sh.at[pl.ds(zbase + q * ZB, ZB)])
    pltpu.sync_copy(zbuf.at[pl.ds(0, 1504)],
                    acc_sh.at[pl.ds(zbase + 3 * ZB, 1504)])

    pltpu.sync_copy(ni_hbm.at[pl.ds(0, N)], ni_ts)
    plsc.subcore_barrier()

    def in_copies(n, b):
        off = wid * WPT + n * CHUNK
        return (
            pltpu.make_async_copy(tgt_hbm.at[pl.ds(off, CHUNK)],
                                  tgtbuf[b], sem_in[b]),
            pltpu.make_async_copy(src_hbm.at[wid * NCHUNK + n],
                                  srcbuf[b], sem_in[b]),
        )

    def gather(b):
        @plsc.parallel_loop(0, R, unroll=4)
        def _row(j):
            for k in range(8):
                idx = tgtbuf[b][pl.ds(j * 128 + k * 16, 16)]
                vals[b][pl.ds(j * 128 + k * 16, 16)] = plsc.load_gather(ni_ts, [idx])

    def fire_scatters(b):
        for j in range(R):
            pltpu.async_copy(vals[b].at[pl.ds(j * 128, 128)],
                             acc_sh.at[srcbuf[b].at[j]],
                             sem_sc[b], add=True)

    def drain_scatters(b):
        for j in range(R):
            pltpu.make_async_copy(vals[b].at[pl.ds(j * 128, 128)],
                                  acc_sh.at[srcbuf[b].at[j]],
                                  sem_sc[b]).wait()

    def start_in(n, b):
        for cp in in_copies(n, b):
            cp.start()

    def wait_in(n, b):
        for cp in in_copies(n, b):
            cp.wait()

    start_in(0, 0)
    start_in(1, 1)

    @pl.loop(0, NCHUNK // 3)
    def _trip(p):
        for b in range(3):
            s0 = b % 3
            s2 = (b + 1) % 3
            n = p * 3 + b
            wait_in(n, s0)
            if b < 2:
                @pl.when(p > 0)
                def _():
                    drain_scatters(s2)
                if b == 0:
                    @pl.when(p > 0)
                    def _():
                        start_in(n + 1, s2)
                else:
                    start_in(n + 1, s2)
            else:
                drain_scatters(s2)
                start_in(n + 1, s2)
            gather(s0)
            fire_scatters(s0)

    ne = NCHUNK - 1
    wait_in(ne, 0)
    drain_scatters(1)
    gather(0)
    fire_scatters(0)

    drain_scatters(2)
    drain_scatters(0)

    plsc.subcore_barrier()
    opt = NPAD // 16
    base = tid * opt
    for q in range(3):
        pltpu.sync_copy(acc_sh.at[pl.ds(base + q * ZB, ZB)], zbuf)
        pltpu.sync_copy(zbuf,
                        out_hbm.at[pl.ds(cid * NPAD + base + q * ZB, ZB)])
    pltpu.sync_copy(acc_sh.at[pl.ds(base + 3 * ZB, 1456)],
                    zbuf.at[pl.ds(0, 1456)])
    pltpu.sync_copy(zbuf.at[pl.ds(0, 1456)],
                    out_hbm.at[pl.ds(cid * NPAD + base + 3 * ZB, 1456)])


def _tc_norm(ni_pad, acc):
    rows = NPAD // 128

    def body(ni_ref, acc_ref, out_ref):
        norm = jnp.sum(ni_ref[...])
        s = D / norm
        v = (acc_ref[0] + acc_ref[1]) * s + (1.0 - D) / N
        r = lax.broadcasted_iota(jnp.int32, (rows, 128), 0)
        c = lax.broadcasted_iota(jnp.int32, (rows, 128), 1)
        out_ref[...] = jnp.where(r * 128 + c < N, v, 0.0)

    out = pl.pallas_call(
        body,
        out_shape=jax.ShapeDtypeStruct((rows, 128), jnp.float32),
    )(ni_pad.reshape(rows, 128), acc.reshape(2, rows, 128))
    return out.reshape(NPAD)


def kernel(node_influence, source_indices, target_indices):
    ni = jnp.zeros((NPAD,), jnp.float32).at[:N].set(node_influence)
    pad = EPAD - E
    src_p = jnp.concatenate(
        [source_indices, jnp.full((pad,), TRASH, jnp.int32)]
    ).reshape(EPAD // CHUNK, R, 128)
    tgt_p = jnp.concatenate([target_indices, jnp.zeros((pad,), jnp.int32)])
    for _ in range(NUM_ITER):
        acc = _sc_iter(ni, src_p, tgt_p)
        ni = _tc_norm(ni, acc)
    return ni[:N]
